# baseline (speedup 1.0000x reference)
import functools

import jax
import jax.numpy as jnp
from jax import lax
from jax.experimental import pallas as pl
from jax.experimental.pallas import tpu as pltpu
from jax.experimental.pallas import tpu_sc as plsc

_F = 26
_K = 16
_D = (_F - 1) * _K
_SL = 4
_B = 4096
_VOCAB = 100000
_VB = 2048
_NVB = (_VOCAB + _VB - 1) // _VB

_info = plsc.get_sparse_core_info()
_NC, _NS = _info.num_cores, _info.num_subcores
_NW = _NC * _NS
_BPW = _B // _NW
_NBUF = 4


def _pairs():
    out = []
    for i in range(_F - 1):
        for j in range(i + 1, _F):
            out.append((i, j))
    return out

_PAIRS = _pairs()


def _repack_body(vref, wref, oref):
    x = vref[...].reshape(_D, _VB)
    wrow = wref[...]
    xp = jnp.concatenate(
        [x, wrow, jnp.zeros((512 - _D - 1, _VB), jnp.float32)], axis=0)
    y = xp.T
    oref[...] = y.reshape(_VB, _SL, 128)


def _ffm_body(idx2d_hbm, vt_hbm, b_hbm, out_hbm, idx2d, tot, out_v, b_v, *rest):
    rows = list(rest[:_NBUF])
    sems = list(rest[_NBUF:2 * _NBUF])

    wid = lax.axis_index("s") * _NC + lax.axis_index("c")
    base = wid * _BPW

    pltpu.sync_copy(idx2d_hbm.at[pl.ds(base, _BPW)], idx2d)
    pltpu.sync_copy(b_hbm, b_v)

    def start(item, par):
        ia = idx2d[item, pl.ds(0, 16)]
        ib = idx2d[item, pl.ds(_F - 16, 16)]
        for f in range(_F):
            r = ia[f] if f < 16 else ib[f - (_F - 16)]
            pltpu.make_async_copy(vt_hbm.at[r], rows[par].at[f],
                                  sems[par]).start()

    def wait(item, par):
        pltpu.make_async_copy(vt_hbm.at[pl.ds(0, _F)], rows[par],
                              sems[par]).wait()

    for par in range(_NBUF):
        start(par, par)

    iota = lax.iota(jnp.int32, 16)

    def item_body(g, carry):
        for par in range(_NBUF):
            item = g * _NBUF + par
            wait(item, par)
            r = rows[par]
            accs = [None] * 8
            for t, (i, j) in enumerate(_PAIRS):
                oa = (j - 1) * _K
                ob = i * _K
                a = r[i, oa // 128, pl.ds(oa % 128, _K)]
                bb = r[j, ob // 128, pl.ds(ob % 128, _K)]
                m = t % 8
                prod = a * bb
                accs[m] = prod if accs[m] is None else accs[m] + prod
            for f in range(_F):
                accs[f % 8] = accs[f % 8] + r[f, _D // 128, pl.ds(_D % 128, _K)]
            acc = (((accs[0] + accs[1]) + (accs[2] + accs[3]))
                   + ((accs[4] + accs[5]) + (accs[6] + accs[7])))
            s1 = acc + lax.rev(acc, (0,))
            total = (((s1[0] + s1[1]) + (s1[2] + s1[3]))
                     + ((s1[4] + s1[5]) + (s1[6] + s1[7])))
            tot[item, :] = jnp.full((16,), total, jnp.float32)

            @pl.when(item + _NBUF < _BPW)
            def _():
                start(item + _NBUF, par)
        return carry

    lax.fori_loop(0, _BPW // _NBUF, item_body, None)

    bvec = b_v[...]
    for blk in range(_BPW // 16):
        y = tot[blk * 16, :]
        for lane in range(1, 16):
            y = jnp.where(iota == lane, tot[blk * 16 + lane, :], y)
        out_v[pl.ds(blk * 16, 16)] = 1.0 / (1.0 + jnp.exp(-(y + bvec)))
    pltpu.sync_copy(out_v, out_hbm.at[pl.ds(base, _BPW)])


@jax.jit
def _run(inputs2d, v2, wT, b16):
    vt = pl.pallas_call(
        _repack_body,
        grid=(_NVB,),
        in_specs=[
            pl.BlockSpec((_F - 1, _K, _VB), lambda g: (0, 0, g)),
            pl.BlockSpec((1, _VB), lambda g: (0, g)),
        ],
        out_specs=pl.BlockSpec((_VB, _SL, 128), lambda g: (g, 0, 0)),
        out_shape=jax.ShapeDtypeStruct((_VOCAB, _SL, 128), jnp.float32),
    )(v2, wT)

    mesh = plsc.VectorSubcoreMesh(core_axis_name="c", subcore_axis_name="s")
    scratch = [
        pltpu.VMEM((_BPW, _F), jnp.int32),
        pltpu.VMEM((_BPW, 16), jnp.float32),
        pltpu.VMEM((_BPW,), jnp.float32),
        pltpu.VMEM((16,), jnp.float32),
    ]
    scratch += [pltpu.VMEM((_F, _SL, 128), jnp.float32) for _ in range(_NBUF)]
    scratch += [pltpu.SemaphoreType.DMA for _ in range(_NBUF)]
    kfn = functools.partial(
        pl.kernel,
        mesh=mesh,
        out_type=jax.ShapeDtypeStruct((_B,), jnp.float32),
        scratch_types=scratch,
        compiler_params=pltpu.CompilerParams(use_tc_tiling_on_sc=True),
    )(_ffm_body)
    return kfn(inputs2d, vt, b16)


def kernel(inputs, w, v, b):
    v2 = jnp.transpose(v, (1, 2, 0))
    wT = jnp.transpose(w.astype(jnp.float32), (1, 0))
    b16 = jnp.broadcast_to(b.astype(jnp.float32), (16,))
    return _run(inputs, v2, wT, b16)


# speedup vs baseline: 1.0982x; 1.0982x over previous
import functools

import jax
import jax.numpy as jnp
from jax import lax
from jax.experimental import pallas as pl
from jax.experimental.pallas import tpu as pltpu
from jax.experimental.pallas import tpu_sc as plsc

_F = 26
_K = 16
_D = (_F - 1) * _K
_SL = 4
_B = 4096
_VOCAB = 100000
_VB = 2048
_NVB = (_VOCAB + _VB - 1) // _VB

_info = plsc.get_sparse_core_info()
_NC, _NS = _info.num_cores, _info.num_subcores
_NW = _NC * _NS
_BPW = _B // _NW
_NBUF = 4


def _pairs():
    out = []
    for i in range(_F - 1):
        for j in range(i + 1, _F):
            out.append((i, j))
    return out

_PAIRS = _pairs()


def _repack_body(vref, wref, oref):
    x = vref[...].reshape(_D, _VB)
    wrow = wref[...]
    xp = jnp.concatenate(
        [x, wrow, jnp.zeros((512 - _D - 1, _VB), jnp.float32)], axis=0)
    y = xp.T
    oref[...] = y.reshape(_VB, _SL, 128)


def _ffm_body(idx2d_hbm, vt_hbm, b_hbm, out_hbm, idx2d, tot, out_v, b_v, *rest):
    rows = list(rest[:_NBUF])
    sems = list(rest[_NBUF:2 * _NBUF])
    sems2 = list(rest[2 * _NBUF:3 * _NBUF])

    wid = lax.axis_index("s") * _NC + lax.axis_index("c")
    base = wid * _BPW

    pltpu.sync_copy(idx2d_hbm.at[pl.ds(base, _BPW)], idx2d)
    pltpu.sync_copy(b_hbm, b_v)

    def start(item, par):
        pltpu.make_async_copy(vt_hbm.at[idx2d.at[item, pl.ds(0, 16)]],
                              rows[par].at[pl.ds(0, 16)],
                              sems[par]).start()
        pltpu.make_async_copy(vt_hbm.at[idx2d.at[item, pl.ds(16, _F - 16)]],
                              rows[par].at[pl.ds(16, _F - 16)],
                              sems2[par]).start()

    def wait(item, par):
        pltpu.make_async_copy(vt_hbm.at[idx2d.at[item, pl.ds(0, 16)]],
                              rows[par].at[pl.ds(0, 16)],
                              sems[par]).wait()
        pltpu.make_async_copy(vt_hbm.at[idx2d.at[item, pl.ds(16, _F - 16)]],
                              rows[par].at[pl.ds(16, _F - 16)],
                              sems2[par]).wait()

    for par in range(_NBUF):
        start(par, par)

    iota = lax.iota(jnp.int32, 16)

    def item_body(g, carry):
        for par in range(_NBUF):
            item = g * _NBUF + par
            wait(item, par)
            r = rows[par]
            accs = [None] * 8
            for t, (i, j) in enumerate(_PAIRS):
                oa = (j - 1) * _K
                ob = i * _K
                a = r[i, oa // 128, pl.ds(oa % 128, _K)]
                bb = r[j, ob // 128, pl.ds(ob % 128, _K)]
                m = t % 8
                prod = a * bb
                accs[m] = prod if accs[m] is None else accs[m] + prod
            for f in range(_F):
                accs[f % 8] = accs[f % 8] + r[f, _D // 128, pl.ds(_D % 128, _K)]
            acc = (((accs[0] + accs[1]) + (accs[2] + accs[3]))
                   + ((accs[4] + accs[5]) + (accs[6] + accs[7])))
            s1 = acc + lax.rev(acc, (0,))
            total = (((s1[0] + s1[1]) + (s1[2] + s1[3]))
                     + ((s1[4] + s1[5]) + (s1[6] + s1[7])))
            tot[item, :] = jnp.full((16,), total, jnp.float32)

            @pl.when(item + _NBUF < _BPW)
            def _():
                start(item + _NBUF, par)
        return carry

    lax.fori_loop(0, _BPW // _NBUF, item_body, None)

    bvec = b_v[...]
    for blk in range(_BPW // 16):
        y = tot[blk * 16, :]
        for lane in range(1, 16):
            y = jnp.where(iota == lane, tot[blk * 16 + lane, :], y)
        out_v[pl.ds(blk * 16, 16)] = 1.0 / (1.0 + jnp.exp(-(y + bvec)))
    pltpu.sync_copy(out_v, out_hbm.at[pl.ds(base, _BPW)])


@jax.jit
def _run(inputs2d, v2, wT, b16):
    vt = pl.pallas_call(
        _repack_body,
        grid=(_NVB,),
        in_specs=[
            pl.BlockSpec((_F - 1, _K, _VB), lambda g: (0, 0, g)),
            pl.BlockSpec((1, _VB), lambda g: (0, g)),
        ],
        out_specs=pl.BlockSpec((_VB, _SL, 128), lambda g: (g, 0, 0)),
        out_shape=jax.ShapeDtypeStruct((_VOCAB, _SL, 128), jnp.float32),
    )(v2, wT)

    mesh = plsc.VectorSubcoreMesh(core_axis_name="c", subcore_axis_name="s")
    scratch = [
        pltpu.VMEM((_BPW, _F), jnp.int32),
        pltpu.VMEM((_BPW, 16), jnp.float32),
        pltpu.VMEM((_BPW,), jnp.float32),
        pltpu.VMEM((16,), jnp.float32),
    ]
    scratch += [pltpu.VMEM((_F, _SL, 128), jnp.float32) for _ in range(_NBUF)]
    scratch += [pltpu.SemaphoreType.DMA for _ in range(2 * _NBUF)]
    kfn = functools.partial(
        pl.kernel,
        mesh=mesh,
        out_type=jax.ShapeDtypeStruct((_B,), jnp.float32),
        scratch_types=scratch,
        compiler_params=pltpu.CompilerParams(use_tc_tiling_on_sc=True),
    )(_ffm_body)
    return kfn(inputs2d, vt, b16)


def kernel(inputs, w, v, b):
    v2 = jnp.transpose(v, (1, 2, 0))
    wT = jnp.transpose(w.astype(jnp.float32), (1, 0))
    b16 = jnp.broadcast_to(b.astype(jnp.float32), (16,))
    return _run(inputs, v2, wT, b16)


# speedup vs baseline: 1.1171x; 1.0172x over previous
import functools

import jax
import jax.numpy as jnp
from jax import lax
from jax.experimental import pallas as pl
from jax.experimental.pallas import tpu as pltpu
from jax.experimental.pallas import tpu_sc as plsc

_F = 26
_K = 16
_D = (_F - 1) * _K
_SL = 4
_B = 4096
_VOCAB = 100000
_VB = 4096
_NVB = (_VOCAB + _VB - 1) // _VB

_info = plsc.get_sparse_core_info()
_NC, _NS = _info.num_cores, _info.num_subcores
_NW = _NC * _NS
_BPW = _B // _NW
_NBUF = 4


def _pairs():
    out = []
    for i in range(_F - 1):
        for j in range(i + 1, _F):
            out.append((i, j))
    return out

_PAIRS = _pairs()


def _repack_body(vref, wref, oref):
    x = vref[...].reshape(_D, _VB)
    wrow = wref[...]
    xp = jnp.concatenate(
        [x, wrow, jnp.zeros((512 - _D - 1, _VB), jnp.float32)], axis=0)
    y = xp.T
    oref[...] = y.reshape(_VB, _SL, 128)


def _ffm_body(idx2d_hbm, vt_hbm, b_hbm, out_hbm, idx2d, tot, out_v, b_v, *rest):
    rows = list(rest[:_NBUF])
    sems = list(rest[_NBUF:2 * _NBUF])

    wid = lax.axis_index("s") * _NC + lax.axis_index("c")
    base = wid * _BPW

    pltpu.sync_copy(idx2d_hbm.at[pl.ds(base, _BPW)], idx2d)
    pltpu.sync_copy(b_hbm, b_v)

    def start(item, par):
        pltpu.make_async_copy(vt_hbm.at[idx2d.at[item]], rows[par],
                              sems[par]).start()

    def wait(item, par):
        pltpu.make_async_copy(vt_hbm.at[idx2d.at[item]], rows[par],
                              sems[par]).wait()

    for par in range(_NBUF):
        start(par, par)

    iota = lax.iota(jnp.int32, 16)

    def item_body(g, carry):
        for par in range(_NBUF):
            item = g * _NBUF + par
            wait(item, par)
            r = rows[par]
            accs = [None] * 8
            for t, (i, j) in enumerate(_PAIRS):
                oa = (j - 1) * _K
                ob = i * _K
                a = r[i, oa // 128, pl.ds(oa % 128, _K)]
                bb = r[j, ob // 128, pl.ds(ob % 128, _K)]
                m = t % 8
                prod = a * bb
                accs[m] = prod if accs[m] is None else accs[m] + prod
            for f in range(_F):
                accs[f % 8] = accs[f % 8] + r[f, _D // 128, pl.ds(_D % 128, _K)]
            acc = (((accs[0] + accs[1]) + (accs[2] + accs[3]))
                   + ((accs[4] + accs[5]) + (accs[6] + accs[7])))
            s1 = acc + lax.rev(acc, (0,))
            total = (((s1[0] + s1[1]) + (s1[2] + s1[3]))
                     + ((s1[4] + s1[5]) + (s1[6] + s1[7])))
            tot[item, :] = jnp.full((16,), total, jnp.float32)

            @pl.when(item + _NBUF < _BPW)
            def _():
                start(item + _NBUF, par)
        return carry

    lax.fori_loop(0, _BPW // _NBUF, item_body, None)

    bvec = b_v[...]
    for blk in range(_BPW // 16):
        y = tot[blk * 16, :]
        for lane in range(1, 16):
            y = jnp.where(iota == lane, tot[blk * 16 + lane, :], y)
        out_v[pl.ds(blk * 16, 16)] = 1.0 / (1.0 + jnp.exp(-(y + bvec)))
    pltpu.sync_copy(out_v, out_hbm.at[pl.ds(base, _BPW)])


@jax.jit
def _run(inputs2d, v2, wT, b16):
    vt = pl.pallas_call(
        _repack_body,
        grid=(_NVB,),
        in_specs=[
            pl.BlockSpec((_F - 1, _K, _VB), lambda g: (0, 0, g)),
            pl.BlockSpec((1, _VB), lambda g: (0, g)),
        ],
        out_specs=pl.BlockSpec((_VB, _SL, 128), lambda g: (g, 0, 0)),
        out_shape=jax.ShapeDtypeStruct((_VOCAB, _SL, 128), jnp.float32),
    )(v2, wT)

    mesh = plsc.VectorSubcoreMesh(core_axis_name="c", subcore_axis_name="s")
    scratch = [
        pltpu.VMEM((_BPW, _F), jnp.int32),
        pltpu.VMEM((_BPW, 16), jnp.float32),
        pltpu.VMEM((_BPW,), jnp.float32),
        pltpu.VMEM((16,), jnp.float32),
    ]
    scratch += [pltpu.VMEM((_F, _SL, 128), jnp.float32) for _ in range(_NBUF)]
    scratch += [pltpu.SemaphoreType.DMA for _ in range(_NBUF)]
    kfn = functools.partial(
        pl.kernel,
        mesh=mesh,
        out_type=jax.ShapeDtypeStruct((_B,), jnp.float32),
        scratch_types=scratch,
        compiler_params=pltpu.CompilerParams(use_tc_tiling_on_sc=True),
    )(_ffm_body)
    return kfn(inputs2d, vt, b16)


def kernel(inputs, w, v, b):
    v2 = jnp.transpose(v, (1, 2, 0))
    wT = jnp.transpose(w.astype(jnp.float32), (1, 0))
    b16 = jnp.broadcast_to(b.astype(jnp.float32), (16,))
    return _run(inputs, v2, wT, b16)
